# baseline (speedup 1.0000x reference)
import jax
import jax.numpy as jnp
from jax.experimental import pallas as pl

B, N, S, SD = 64, 16, 16, 256
HID, EMB, AD, K = 512, 256, 64, 1024

BT = 8
NPROG = B // BT
TOK = N * S
PAIR = (N - 1) * S


def _fused_kernel(slots_ref, w1_ref, b1_ref, g_ref, bln_ref, w2_ref, b2_ref,
                  wm_ref, bm_ref, wv_ref, bv_ref, cb_ref, noise_ref,
                  dist_ref, z_ref, var_ref, proto_ref, idx_ref, ql_ref):
    i = pl.program_id(0)
    f32 = jnp.float32

    x = slots_ref[...].reshape(BT * TOK, SD)
    h = jnp.maximum(jnp.dot(x, w1_ref[...], preferred_element_type=f32)
                    + b1_ref[...], 0.0)
    mu = jnp.mean(h, axis=-1, keepdims=True)
    var = jnp.mean((h - mu) ** 2, axis=-1, keepdims=True)
    h = (h - mu) / jnp.sqrt(var + 1e-5) * g_ref[...] + bln_ref[...]
    emb = jnp.dot(h, w2_ref[...], preferred_element_type=f32) + b2_ref[...]

    wmv = jnp.concatenate([wm_ref[...], wv_ref[...]], axis=1)
    bmv = jnp.concatenate([bm_ref[...], bv_ref[...]], axis=0)
    mv = jnp.dot(emb, wmv, preferred_element_type=f32) + bmv
    mt = mv[:, :AD]
    vt = jnp.abs(mv[:, AD:])

    mt3 = mt.reshape(BT, TOK, AD)
    vt3 = vt.reshape(BT, TOK, AD)
    dm = mt3[:, S:, :] - mt3[:, :-S, :]
    dv = vt3[:, S:, :] + vt3[:, :-S, :]

    noise = noise_ref[...].reshape(BT, PAIR, AD)
    z = noise * jnp.sqrt(dv + 1e-6) + dm
    zf = z.reshape(BT * PAIR, AD)

    cb = cb_ref[...]
    cross = jax.lax.dot_general(zf, cb, (((1,), (1,)), ((), ())),
                                preferred_element_type=f32)
    d = (jnp.sum(zf * zf, axis=1, keepdims=True) - 2.0 * cross
         + jnp.sum(cb * cb, axis=1)[None, :])

    idx = jnp.argmin(d, axis=1).astype(jnp.int32)
    iota = jax.lax.broadcasted_iota(jnp.int32, d.shape, 1)
    onehot = (iota == idx[:, None]).astype(f32)
    q = jnp.dot(onehot, cb, preferred_element_type=f32)

    resid = zf - q

    dist_ref[:, :, 0] = dm.reshape(BT, N - 1, S, AD)
    dist_ref[:, :, 1] = dv.reshape(BT, N - 1, S, AD)
    z_ref[...] = z.reshape(BT, N - 1, S, AD)
    var_ref[...] = resid.reshape(BT, N - 1, S, AD)
    proto_ref[...] = q.reshape(BT, N - 1, S, AD)
    idx_ref[...] = idx.reshape(BT, N - 1, S, 1)

    partial = jnp.sum(resid * resid).reshape(1, 1)

    @pl.when(i == 0)
    def _():
        ql_ref[...] = jnp.zeros((1, 1), f32)

    ql_ref[...] += partial

    @pl.when(i == NPROG - 1)
    def _():
        ql_ref[...] = ql_ref[...] * (1.0 / (B * PAIR * AD))


@jax.jit
def _run(slots, W1, b1, ln_g, ln_b, W2, b2, Wm, bm, Wv, bv, codebook, noise):
    out_shapes = (
        jax.ShapeDtypeStruct((B, N - 1, 2, S, AD), jnp.float32),
        jax.ShapeDtypeStruct((B, N - 1, S, AD), jnp.float32),
        jax.ShapeDtypeStruct((B, N - 1, S, AD), jnp.float32),
        jax.ShapeDtypeStruct((B, N - 1, S, AD), jnp.float32),
        jax.ShapeDtypeStruct((B, N - 1, S, 1), jnp.int32),
        jax.ShapeDtypeStruct((1, 1), jnp.float32),
    )
    full = lambda shape: pl.BlockSpec(shape, lambda i: tuple(0 for _ in shape))
    in_specs = [
        pl.BlockSpec((BT, N, S, SD), lambda i: (i, 0, 0, 0)),
        full(W1.shape), full(b1.shape), full(ln_g.shape), full(ln_b.shape),
        full(W2.shape), full(b2.shape), full(Wm.shape), full(bm.shape),
        full(Wv.shape), full(bv.shape), full(codebook.shape),
        pl.BlockSpec((BT, N - 1, S, AD), lambda i: (i, 0, 0, 0)),
    ]
    out_specs = (
        pl.BlockSpec((BT, N - 1, 2, S, AD), lambda i: (i, 0, 0, 0, 0)),
        pl.BlockSpec((BT, N - 1, S, AD), lambda i: (i, 0, 0, 0)),
        pl.BlockSpec((BT, N - 1, S, AD), lambda i: (i, 0, 0, 0)),
        pl.BlockSpec((BT, N - 1, S, AD), lambda i: (i, 0, 0, 0)),
        pl.BlockSpec((BT, N - 1, S, 1), lambda i: (i, 0, 0, 0)),
        pl.BlockSpec((1, 1), lambda i: (0, 0)),
    )
    dist, z, variability, protos, idxs, ql = pl.pallas_call(
        _fused_kernel,
        grid=(NPROG,),
        in_specs=in_specs,
        out_specs=out_specs,
        out_shape=out_shapes,
    )(slots, W1, b1, ln_g, ln_b, W2, b2, Wm, bm, Wv, bv, codebook, noise)
    loss = ql.reshape(())
    return (dist, z, variability, protos, idxs, loss, loss)


_NOISE_CACHE = []


def _noise():
    if not _NOISE_CACHE:
        _NOISE_CACHE.append(jax.random.normal(
            jax.random.key(42), (B, N - 1, S, AD), dtype=jnp.float32))
    return _NOISE_CACHE[0]


def kernel(slots, W1, b1, ln_g, ln_b, W2, b2, Wm, bm, Wv, bv, codebook):
    return _run(slots, W1, b1, ln_g, ln_b, W2, b2, Wm, bm, Wv, bv, codebook,
                _noise())


# speedup vs baseline: 1.1684x; 1.1684x over previous
import jax
import jax.numpy as jnp
from jax.experimental import pallas as pl
from jax.experimental.pallas import tpu as pltpu

B, N, S, SD = 64, 16, 16, 256
HID, EMB, AD, K = 512, 256, 64, 1024

BT = 8
NPROG = B // BT
TOK = N * S
PAIR = (N - 1) * S


def _fused_kernel(slots_ref, w1_ref, b1_ref, g_ref, bln_ref, w2_ref, b2_ref,
                  wmv_ref, bmv_ref, cb_ref, noise_ref,
                  dist_ref, z_ref, var_ref, proto_ref, idx_ref, ql_ref):
    f32 = jnp.float32

    x = slots_ref[...].reshape(BT * TOK, SD)
    h = jnp.maximum(jnp.dot(x, w1_ref[...], preferred_element_type=f32)
                    + b1_ref[...], 0.0)
    mu = jnp.mean(h, axis=-1, keepdims=True)
    var = jnp.mean((h - mu) ** 2, axis=-1, keepdims=True)
    h = (h - mu) / jnp.sqrt(var + 1e-5) * g_ref[...] + bln_ref[...]
    emb = jnp.dot(h, w2_ref[...], preferred_element_type=f32) + b2_ref[...]

    mv = jnp.dot(emb, wmv_ref[...], preferred_element_type=f32) + bmv_ref[...]
    mt = mv[:, :AD]
    vt = jnp.abs(mv[:, AD:])

    mt3 = mt.reshape(BT, TOK, AD)
    vt3 = vt.reshape(BT, TOK, AD)
    dm = mt3[:, S:, :] - mt3[:, :-S, :]
    dv = vt3[:, S:, :] + vt3[:, :-S, :]

    noise = noise_ref[...].reshape(BT, PAIR, AD)
    z = noise * jnp.sqrt(dv + 1e-6) + dm
    zf = z.reshape(BT * PAIR, AD)

    cb = cb_ref[...]
    cross = jax.lax.dot_general(zf, cb, (((1,), (1,)), ((), ())),
                                preferred_element_type=f32)
    d = -2.0 * cross + jnp.sum(cb * cb, axis=1)[None, :]

    idx = jnp.argmin(d, axis=1).astype(jnp.int32)
    iota = jax.lax.broadcasted_iota(jnp.int32, d.shape, 1)
    onehot = (iota == idx[:, None]).astype(f32)
    q = jnp.dot(onehot, cb, preferred_element_type=f32)

    resid = zf - q

    dist_ref[:, :, 0] = dm.reshape(BT, N - 1, S, AD)
    dist_ref[:, :, 1] = dv.reshape(BT, N - 1, S, AD)
    z_ref[...] = z.reshape(BT, N - 1, S, AD)
    var_ref[...] = resid.reshape(BT, N - 1, S, AD)
    proto_ref[...] = q.reshape(BT, N - 1, S, AD)
    idx_ref[0, 0, :] = idx
    ql_ref[...] = jnp.sum(resid * resid).reshape(1, 1, 1)


@jax.jit
def _run(slots, W1, b1, ln_g, ln_b, W2, b2, Wmv, bmv, codebook, noise):
    out_shapes = (
        jax.ShapeDtypeStruct((B, N - 1, 2, S, AD), jnp.float32),
        jax.ShapeDtypeStruct((B, N - 1, S, AD), jnp.float32),
        jax.ShapeDtypeStruct((B, N - 1, S, AD), jnp.float32),
        jax.ShapeDtypeStruct((B, N - 1, S, AD), jnp.float32),
        jax.ShapeDtypeStruct((NPROG, 1, BT * PAIR), jnp.int32),
        jax.ShapeDtypeStruct((NPROG, 1, 1), jnp.float32),
    )
    full = lambda shape: pl.BlockSpec(shape, lambda i: tuple(0 for _ in shape))
    in_specs = [
        pl.BlockSpec((BT, N, S, SD), lambda i: (i, 0, 0, 0)),
        full(W1.shape), full(b1.shape), full(ln_g.shape), full(ln_b.shape),
        full(W2.shape), full(b2.shape), full(Wmv.shape), full(bmv.shape),
        full(codebook.shape),
        pl.BlockSpec((BT, N - 1, S, AD), lambda i: (i, 0, 0, 0)),
    ]
    out_specs = (
        pl.BlockSpec((BT, N - 1, 2, S, AD), lambda i: (i, 0, 0, 0, 0)),
        pl.BlockSpec((BT, N - 1, S, AD), lambda i: (i, 0, 0, 0)),
        pl.BlockSpec((BT, N - 1, S, AD), lambda i: (i, 0, 0, 0)),
        pl.BlockSpec((BT, N - 1, S, AD), lambda i: (i, 0, 0, 0)),
        pl.BlockSpec((1, 1, BT * PAIR), lambda i: (i, 0, 0)),
        pl.BlockSpec((1, 1, 1), lambda i: (i, 0, 0)),
    )
    dist, z, variability, protos, idx_flat, ql = pl.pallas_call(
        _fused_kernel,
        grid=(NPROG,),
        in_specs=in_specs,
        out_specs=out_specs,
        out_shape=out_shapes,
        compiler_params=pltpu.CompilerParams(
            dimension_semantics=("parallel",)),
    )(slots, W1, b1, ln_g, ln_b, W2, b2, Wmv, bmv, codebook, noise)
    action_idxs = idx_flat.reshape(B, N - 1, S, 1)
    loss = (jnp.sum(ql) / (B * PAIR * AD)).reshape(())
    return (dist, z, variability, protos, action_idxs, loss, loss)


_NOISE_CACHE = []


def _noise():
    if not _NOISE_CACHE:
        _NOISE_CACHE.append(jax.random.normal(
            jax.random.key(42), (B, N - 1, S, AD), dtype=jnp.float32))
    return _NOISE_CACHE[0]


def kernel(slots, W1, b1, ln_g, ln_b, W2, b2, Wm, bm, Wv, bv, codebook):
    Wmv = jnp.concatenate([Wm, Wv], axis=1)
    bmv = jnp.concatenate([bm, bv], axis=0)
    return _run(slots, W1, b1, ln_g, ln_b, W2, b2, Wmv, bmv, codebook,
                _noise())


# speedup vs baseline: 1.2042x; 1.0307x over previous
import jax
import jax.numpy as jnp
from jax.experimental import pallas as pl
from jax.experimental.pallas import tpu as pltpu

B, N, S, SD = 64, 16, 16, 256
HID, EMB, AD, K = 512, 256, 64, 1024

BT = 8
NPROG = B // BT
TOK = N * S
PAIR = (N - 1) * S


def _fused_kernel(slots_ref, w1_ref, b1_ref, g_ref, bln_ref, w2_ref, b2_ref,
                  wmv_ref, bmv_ref, cb_ref, cba_ref, noise_ref,
                  dist_ref, z_ref, var_ref, proto_ref, idx_ref, ql_ref):
    f32 = jnp.float32

    x = slots_ref[...].reshape(BT * TOK, SD)
    h = jnp.maximum(jnp.dot(x, w1_ref[...], preferred_element_type=f32)
                    + b1_ref[...], 0.0)
    mu = jnp.mean(h, axis=-1, keepdims=True)
    var = jnp.mean(h * h, axis=-1, keepdims=True) - mu * mu
    h = (h - mu) / jnp.sqrt(var + 1e-5) * g_ref[...] + bln_ref[...]
    emb = jnp.dot(h, w2_ref[...], preferred_element_type=f32) + b2_ref[...]

    mv = jnp.dot(emb, wmv_ref[...], preferred_element_type=f32) + bmv_ref[...]
    mt = mv[:, :AD]
    vt = jnp.abs(mv[:, AD:])

    mt3 = mt.reshape(BT, TOK, AD)
    vt3 = vt.reshape(BT, TOK, AD)
    dm = mt3[:, S:, :] - mt3[:, :-S, :]
    dv = vt3[:, S:, :] + vt3[:, :-S, :]

    noise = noise_ref[...].reshape(BT, PAIR, AD)
    z = noise * jnp.sqrt(dv + 1e-6) + dm
    zf = z.reshape(BT * PAIR, AD)

    cb = cb_ref[...]
    zfa = jnp.concatenate([zf, jnp.ones((BT * PAIR, 1), f32)], axis=1)
    d = jax.lax.dot_general(zfa, cba_ref[...], (((1,), (1,)), ((), ())),
                            preferred_element_type=f32)

    idx = jnp.argmin(d, axis=1).astype(jnp.int32)
    iota = jax.lax.broadcasted_iota(jnp.int32, d.shape, 1)
    onehot = (iota == idx[:, None]).astype(f32)
    q = jnp.dot(onehot, cb, preferred_element_type=f32)

    resid = zf - q

    dist_ref[:, :, 0] = dm.reshape(BT, N - 1, S, AD)
    dist_ref[:, :, 1] = dv.reshape(BT, N - 1, S, AD)
    z_ref[...] = z.reshape(BT, N - 1, S, AD)
    var_ref[...] = resid.reshape(BT, N - 1, S, AD)
    proto_ref[...] = q.reshape(BT, N - 1, S, AD)
    idx_ref[0, 0, :] = idx
    ql_ref[...] = jnp.sum(resid * resid).reshape(1, 1, 1)


@jax.jit
def _run(slots, W1, b1, ln_g, ln_b, W2, b2, Wmv, bmv, codebook, cba, noise):
    out_shapes = (
        jax.ShapeDtypeStruct((B, N - 1, 2, S, AD), jnp.float32),
        jax.ShapeDtypeStruct((B, N - 1, S, AD), jnp.float32),
        jax.ShapeDtypeStruct((B, N - 1, S, AD), jnp.float32),
        jax.ShapeDtypeStruct((B, N - 1, S, AD), jnp.float32),
        jax.ShapeDtypeStruct((NPROG, 1, BT * PAIR), jnp.int32),
        jax.ShapeDtypeStruct((NPROG, 1, 1), jnp.float32),
    )
    full = lambda shape: pl.BlockSpec(shape, lambda i: tuple(0 for _ in shape))
    in_specs = [
        pl.BlockSpec((BT, N, S, SD), lambda i: (i, 0, 0, 0)),
        full(W1.shape), full(b1.shape), full(ln_g.shape), full(ln_b.shape),
        full(W2.shape), full(b2.shape), full(Wmv.shape), full(bmv.shape),
        full(codebook.shape), full(cba.shape),
        pl.BlockSpec((BT, N - 1, S, AD), lambda i: (i, 0, 0, 0)),
    ]
    out_specs = (
        pl.BlockSpec((BT, N - 1, 2, S, AD), lambda i: (i, 0, 0, 0, 0)),
        pl.BlockSpec((BT, N - 1, S, AD), lambda i: (i, 0, 0, 0)),
        pl.BlockSpec((BT, N - 1, S, AD), lambda i: (i, 0, 0, 0)),
        pl.BlockSpec((BT, N - 1, S, AD), lambda i: (i, 0, 0, 0)),
        pl.BlockSpec((1, 1, BT * PAIR), lambda i: (i, 0, 0)),
        pl.BlockSpec((1, 1, 1), lambda i: (i, 0, 0)),
    )
    dist, z, variability, protos, idx_flat, ql = pl.pallas_call(
        _fused_kernel,
        grid=(NPROG,),
        in_specs=in_specs,
        out_specs=out_specs,
        out_shape=out_shapes,
        compiler_params=pltpu.CompilerParams(
            dimension_semantics=("parallel",)),
    )(slots, W1, b1, ln_g, ln_b, W2, b2, Wmv, bmv, codebook, cba, noise)
    action_idxs = idx_flat.reshape(B, N - 1, S, 1)
    loss = (jnp.sum(ql) / (B * PAIR * AD)).reshape(())
    return (dist, z, variability, protos, action_idxs, loss, loss)


_NOISE_CACHE = []


def _noise():
    if not _NOISE_CACHE:
        _NOISE_CACHE.append(jax.random.normal(
            jax.random.key(42), (B, N - 1, S, AD), dtype=jnp.float32))
    return _NOISE_CACHE[0]


def kernel(slots, W1, b1, ln_g, ln_b, W2, b2, Wm, bm, Wv, bv, codebook):
    Wmv = jnp.concatenate([Wm, Wv], axis=1)
    bmv = jnp.concatenate([bm, bv], axis=0)
    cba = jnp.concatenate(
        [codebook * -2.0, jnp.sum(codebook * codebook, axis=1, keepdims=True)],
        axis=1)
    return _run(slots, W1, b1, ln_g, ln_b, W2, b2, Wmv, bmv, codebook, cba,
                _noise())
